# Initial kernel scaffold; baseline (speedup 1.0000x reference)
import jax
import jax.numpy as jnp
from jax.experimental import pallas as pl


def kernel(data, labels, org_edge_index, emb_table, W_lin, att_i, att_j, att_em_i, att_em_j, gnn_bias, bn1_gamma, bn1_beta, bn2_gamma, bn2_beta, out_W, out_b):
    raise NotImplementedError("write your pallas kernel here")


# speedup vs baseline: 50.0810x; 50.0810x over previous
import functools

import jax
import jax.numpy as jnp
from jax import lax
from jax.experimental import pallas as pl
from jax.experimental.pallas import tpu as pltpu
from jax.experimental.pallas import tpu_sc as plsc

_B, _N, _F, _D = 128, 100, 64, 64
_G = 4
_GN = _G * _N
_NGROUPS = _B // _G
_NEDGE = 1600
_LANES = 16


def _counts_body(edges_hbm, out_hbm, src_v, dst_v, acc_v):
    c = lax.axis_index("c")
    s = lax.axis_index("s")

    @pl.when(jnp.logical_and(c == 0, s == 0))
    def _():
        pltpu.sync_copy(edges_hbm.at[0], src_v)
        pltpu.sync_copy(edges_hbm.at[1], dst_v)

        def zero_body(i, carry):
            acc_v[pl.ds(i * _LANES, _LANES)] = jnp.zeros((_LANES,), jnp.float32)
            return carry

        lax.fori_loop(0, (_N * _N) // _LANES, zero_body, 0)

        ones = jnp.full((_LANES,), 1.0, jnp.float32)

        def edge_body(i, carry):
            sv = src_v[pl.ds(i * _LANES, _LANES)]
            dv = dst_v[pl.ds(i * _LANES, _LANES)]
            idx = dv * _N + sv
            plsc.addupdate_scatter(acc_v, [idx], ones)
            return carry

        lax.fori_loop(0, _NEDGE // _LANES, edge_body, 0)
        pltpu.sync_copy(acc_v, out_hbm)


_build_counts = functools.partial(
    pl.kernel,
    mesh=plsc.VectorSubcoreMesh(core_axis_name="c", subcore_axis_name="s"),
    out_type=jax.ShapeDtypeStruct((_N * _N,), jnp.float32),
    compiler_params=pltpu.CompilerParams(needs_layout_passes=False),
    scratch_types=[
        pltpu.VMEM((_NEDGE,), jnp.int32),
        pltpu.VMEM((_NEDGE,), jnp.int32),
        pltpu.VMEM((_N * _N,), jnp.float32),
    ],
)(_counts_body)


def _mm(a, b, ca, cb):
    return lax.dot_general(a, b, ((( ca,), (cb,)), ((), ())),
                           precision=lax.Precision.HIGHEST,
                           preferred_element_type=jnp.float32)


def _fused_body(x_ref, emb4_ref, cblk_ref, wlin_ref, ai_ref, aj_ref, aei_ref,
                aej_ref, bias_ref, g1_ref, b1_ref, g2_ref, b2_ref, ow_ref,
                ob_ref, o_ref, out_s):
    wlin = wlin_ref[:]
    em_g = emb4_ref[:]
    att_i = ai_ref[:]
    att_j = aj_ref[:]
    att_em_i = aei_ref[:]
    att_em_j = aej_ref[:]
    bias = bias_ref[:]
    cblk = cblk_ref[:]

    r_id = lax.broadcasted_iota(jnp.int32, (_GN, _GN), 0)
    c_id = lax.broadcasted_iota(jnp.int32, (_GN, _GN), 1)
    eye = (r_id == c_id).astype(jnp.float32)

    zstat = jnp.zeros((1, _D), jnp.float32)

    ei = _mm(em_g, att_em_i, 1, 1)
    ej = _mm(em_g, att_em_j, 1, 1)

    def group_body(g, carry):
        s1, s2 = carry
        base = g * _GN
        wx_g = _mm(x_ref[pl.ds(base, _GN), :], wlin, 1, 1)
        ai = _mm(wx_g, att_i, 1, 1) + ei
        aj_c = _mm(wx_g, att_j, 1, 1) + ej
        aj_r = _mm(aj_c, eye, 0, 0)
        alpha = ai + aj_r
        alpha = jnp.where(alpha >= 0, alpha, 0.2 * alpha)
        am = jnp.max(jnp.where(cblk > 0, alpha, -1e30), axis=1, keepdims=True)
        ex = cblk * jnp.exp(jnp.minimum(alpha - am, 0.0))
        den = jnp.sum(ex, axis=1, keepdims=True)
        w = ex / (den + 1e-16)
        out_g = _mm(w, wx_g, 1, 0) + bias
        out_s[pl.ds(base, _GN), :] = out_g
        s1 = s1 + jnp.sum(out_g, axis=0, keepdims=True)
        s2 = s2 + jnp.sum(out_g * out_g, axis=0, keepdims=True)
        return s1, s2

    s1, s2 = lax.fori_loop(0, _NGROUPS, group_body, (zstat, zstat))

    n_rows = float(_B * _N)
    mu1 = s1 / n_rows
    var1 = s2 / n_rows - mu1 * mu1
    scale1 = lax.rsqrt(var1 + 1e-5) * g1_ref[:]
    b1 = b1_ref[:]

    def bn1_body(g, carry):
        t1, t2 = carry
        base = g * _GN
        o1 = (out_s[pl.ds(base, _GN), :] - mu1) * scale1 + b1
        h = jnp.maximum(o1, 0.0) * em_g
        out_s[pl.ds(base, _GN), :] = h
        t1 = t1 + jnp.sum(h, axis=0, keepdims=True)
        t2 = t2 + jnp.sum(h * h, axis=0, keepdims=True)
        return t1, t2

    t1, t2 = lax.fori_loop(0, _NGROUPS, bn1_body, (zstat, zstat))

    mu2 = t1 / n_rows
    var2 = t2 / n_rows - mu2 * mu2
    scale2 = lax.rsqrt(var2 + 1e-5) * g2_ref[:]
    b2 = b2_ref[:]
    ow = ow_ref[:]
    ob = ob_ref[:]

    def bn2_body(g, carry):
        base = g * _GN
        h2 = jnp.maximum((out_s[pl.ds(base, _GN), :] - mu2) * scale2 + b2, 0.0)
        o_ref[pl.ds(base, _GN), :] = _mm(h2, ow, 1, 1) + ob
        return carry

    lax.fori_loop(0, _NGROUPS, bn2_body, 0)


_fused = pl.pallas_call(
    _fused_body,
    out_shape=jax.ShapeDtypeStruct((_B * _N, 8), jnp.float32),
    scratch_shapes=[
        pltpu.VMEM((_B * _N, _D), jnp.float32),
    ],
)


def kernel(data, labels, org_edge_index, emb_table, W_lin, att_i, att_j,
           att_em_i, att_em_j, gnn_bias, bn1_gamma, bn1_beta, bn2_gamma,
           bn2_beta, out_W, out_b):
    B, N, F = data.shape

    loops = jnp.arange(N, dtype=org_edge_index.dtype)
    src_all = jnp.concatenate([org_edge_index[0], loops])
    dst_all = jnp.concatenate([org_edge_index[1], loops])
    edges = jnp.stack([src_all, dst_all])

    counts = _build_counts(edges)
    c_mat = counts.reshape(N, N)
    c_blk = jnp.kron(jnp.eye(_G, dtype=jnp.float32), c_mat)

    x2 = data.reshape(B * N, F)
    emb4 = jnp.tile(emb_table, (_G, 1))

    ow8 = jnp.zeros((8, out_W.shape[1]), jnp.float32).at[0].set(out_W[0])
    ob8 = jnp.zeros((1, 8), jnp.float32).at[0, 0].set(out_b[0])
    o = _fused(x2, emb4, c_blk, W_lin,
               att_i.reshape(1, -1), att_j.reshape(1, -1),
               att_em_i.reshape(1, -1), att_em_j.reshape(1, -1),
               gnn_bias.reshape(1, -1), bn1_gamma.reshape(1, -1),
               bn1_beta.reshape(1, -1), bn2_gamma.reshape(1, -1),
               bn2_beta.reshape(1, -1), ow8, ob8)
    return o[:, 0].reshape(B, N)


# speedup vs baseline: 57.4622x; 1.1474x over previous
import functools

import jax
import jax.numpy as jnp
from jax import lax
from jax.experimental import pallas as pl
from jax.experimental.pallas import tpu as pltpu
from jax.experimental.pallas import tpu_sc as plsc

_B, _N, _F, _D = 128, 100, 64, 64
_G = 4
_GN = _G * _N
_NGROUPS = _B // _G
_CH = 16
_CHN = _CH * _N
_NCHUNK = _B // _CH
_NEDGE = 1500
_LANES = 16


def _counts_body(edges_hbm, out_hbm, src_v, dst_v, acc_v):
    c = lax.axis_index("c")
    s = lax.axis_index("s")

    @pl.when(jnp.logical_and(c == 0, s == 0))
    def _():
        pltpu.sync_copy(edges_hbm.at[0], src_v)
        pltpu.sync_copy(edges_hbm.at[1], dst_v)

        def zero_body(i, carry):
            acc_v[pl.ds(i * _LANES, _LANES)] = jnp.zeros((_LANES,),
                                                         jnp.float32)
            return carry

        lax.fori_loop(0, (_N * _N) // _LANES, zero_body, 0)

        ones = jnp.full((_LANES,), 1.0, jnp.float32)
        lane = lax.iota(jnp.int32, _LANES)

        def edge_body(i, carry):
            sv = src_v[pl.ds(i * _LANES, _LANES)]
            dv = dst_v[pl.ds(i * _LANES, _LANES)]
            valid = (i * _LANES + lane) < _NEDGE
            plsc.addupdate_scatter(acc_v, [dv * _N + sv], ones, mask=valid)
            return carry

        nvec = (_NEDGE + _LANES - 1) // _LANES
        lax.fori_loop(0, nvec, edge_body, 0)

        def loop_body(i, carry):
            node = i * _LANES + lane
            valid = node < _N
            plsc.addupdate_scatter(acc_v, [node * (_N + 1)], ones, mask=valid)
            return carry

        lax.fori_loop(0, (_N + _LANES - 1) // _LANES, loop_body, 0)
        pltpu.sync_copy(acc_v, out_hbm)


_build_counts = functools.partial(
    pl.kernel,
    mesh=plsc.VectorSubcoreMesh(core_axis_name="c", subcore_axis_name="s"),
    out_type=jax.ShapeDtypeStruct((_N * _N,), jnp.float32),
    compiler_params=pltpu.CompilerParams(needs_layout_passes=False),
    scratch_types=[
        pltpu.VMEM((_NEDGE + 4,), jnp.int32),
        pltpu.VMEM((_NEDGE + 4,), jnp.int32),
        pltpu.VMEM((_N * _N,), jnp.float32),
    ],
)(_counts_body)


def _mm(a, b, ca, cb):
    return lax.dot_general(a, b, (((ca,), (cb,)), ((), ())),
                           precision=lax.Precision.HIGHEST,
                           preferred_element_type=jnp.float32)


def _node_mask(rows):
    r_id = lax.broadcasted_iota(jnp.int32, (rows, _N), 0)
    c_id = lax.broadcasted_iota(jnp.int32, (rows, _N), 1)
    return (lax.rem(r_id, _N) == c_id).astype(jnp.float32)


def _stripe_mask(nb, rows):
    q_id = lax.broadcasted_iota(jnp.int32, (nb, rows), 0)
    r_id = lax.broadcasted_iota(jnp.int32, (nb, rows), 1)
    return ((r_id // _N) == q_id).astype(jnp.float32)


def _fused_body(x_ref, emb_ref, c_ref, wlin_ref, ai_ref, aj_ref, aei_ref,
                aej_ref, bias_ref, g1_ref, b1_ref, g2_ref, b2_ref, ow_ref,
                obn_ref, o_ref, out_s):
    wlin = wlin_ref[:]
    att_i = ai_ref[:]
    att_j = aj_ref[:]
    bias = bias_ref[:]

    nm16 = _node_mask(_CHN)
    nm4 = nm16[:_GN, :]
    sel4 = _stripe_mask(_G, _GN)
    sel16 = _stripe_mask(_CH, _CHN)

    emb16 = _mm(nm16, emb_ref[:], 1, 0)
    emb4 = emb16[:_GN, :]
    c4 = _mm(nm4, c_ref[:], 1, 0)
    ei = _mm(emb4, aei_ref[:], 1, 1)
    ej = _mm(emb4, aej_ref[:], 1, 1)

    zstat = jnp.zeros((1, _D), jnp.float32)

    def group_body(g, carry):
        s1, s2 = carry
        base = g * _GN
        wx_g = _mm(x_ref[pl.ds(base, _GN), :], wlin, 1, 1)
        ai = _mm(wx_g, att_i, 1, 1) + ei
        aj_c = _mm(wx_g, att_j, 1, 1) + ej
        ajg = _mm(sel4, aj_c * nm4, 1, 0)
        aj_rep = _mm(sel4, ajg, 0, 0)
        alpha = ai + aj_rep
        alpha = jnp.where(alpha >= 0, alpha, 0.2 * alpha)
        am = jnp.max(jnp.where(c4 > 0, alpha, -1e30), axis=1, keepdims=True)
        ex = c4 * jnp.exp(jnp.minimum(alpha - am, 0.0))
        den = jnp.sum(ex, axis=1, keepdims=True)
        w = ex / (den + 1e-16)
        out_g = jnp.concatenate(
            [_mm(w[q * _N:(q + 1) * _N, :],
                 wx_g[q * _N:(q + 1) * _N, :], 1, 0) for q in range(_G)],
            axis=0) + bias
        out_s[pl.ds(base, _GN), :] = out_g
        s1 = s1 + jnp.sum(out_g, axis=0, keepdims=True)
        s2 = s2 + jnp.sum(out_g * out_g, axis=0, keepdims=True)
        return s1, s2

    s1, s2 = lax.fori_loop(0, _NGROUPS, group_body, (zstat, zstat))

    n_rows = float(_B * _N)
    mu1 = s1 / n_rows
    var1 = s2 / n_rows - mu1 * mu1
    scale1 = lax.rsqrt(var1 + 1e-5) * g1_ref[:]
    b1 = b1_ref[:]

    def bn1_body(i, carry):
        t1, t2 = carry
        base = i * _CHN
        o1 = (out_s[pl.ds(base, _CHN), :] - mu1) * scale1 + b1
        h = jnp.maximum(o1, 0.0) * emb16
        out_s[pl.ds(base, _CHN), :] = h
        t1 = t1 + jnp.sum(h, axis=0, keepdims=True)
        t2 = t2 + jnp.sum(h * h, axis=0, keepdims=True)
        return t1, t2

    t1, t2 = lax.fori_loop(0, _NCHUNK, bn1_body, (zstat, zstat))

    mu2 = t1 / n_rows
    var2 = t2 / n_rows - mu2 * mu2
    scale2 = lax.rsqrt(var2 + 1e-5) * g2_ref[:]
    b2 = b2_ref[:]
    ow = ow_ref[:]
    obn = obn_ref[:]

    def bn2_body(i, carry):
        base = i * _CHN
        h2 = jnp.maximum((out_s[pl.ds(base, _CHN), :] - mu2) * scale2 + b2,
                         0.0)
        oc = _mm(h2, ow, 1, 1)
        orows = _mm(sel16, oc * nm16, 1, 0)
        o_ref[pl.ds(i * _CH, _CH), :] = orows + obn
        return carry

    lax.fori_loop(0, _NCHUNK, bn2_body, 0)


_fused = pl.pallas_call(
    _fused_body,
    out_shape=jax.ShapeDtypeStruct((_B, _N), jnp.float32),
    scratch_shapes=[
        pltpu.VMEM((_B * _N, _D), jnp.float32),
    ],
)


def kernel(data, labels, org_edge_index, emb_table, W_lin, att_i, att_j,
           att_em_i, att_em_j, gnn_bias, bn1_gamma, bn1_beta, bn2_gamma,
           bn2_beta, out_W, out_b):
    B, N, F = data.shape

    edges = jnp.pad(org_edge_index, ((0, 0), (0, 4)))
    counts = _build_counts(edges)
    c_mat = counts.reshape(N, N)

    x2 = data.reshape(B * N, F)
    obn = jnp.broadcast_to(out_b.reshape(1, 1), (1, N))

    return _fused(x2, emb_table, c_mat, W_lin,
                  att_i.reshape(1, -1), att_j.reshape(1, -1),
                  att_em_i.reshape(1, -1), att_em_j.reshape(1, -1),
                  gnn_bias.reshape(1, -1), bn1_gamma.reshape(1, -1),
                  bn1_beta.reshape(1, -1), bn2_gamma.reshape(1, -1),
                  bn2_beta.reshape(1, -1), out_W.reshape(1, -1), obn)


# speedup vs baseline: 74.6426x; 1.2990x over previous
import functools

import jax
import jax.numpy as jnp
from jax import lax
from jax.experimental import pallas as pl
from jax.experimental.pallas import tpu as pltpu
from jax.experimental.pallas import tpu_sc as plsc

_B, _N, _F, _D = 128, 100, 64, 64
_G = 4
_GN = _G * _N
_NGROUPS = _B // _G
_CH = 16
_CHN = _CH * _N
_NCHUNK = _B // _CH
_NEDGE = 1500
_LANES = 16


def _counts_body(edges_hbm, out_hbm, src_v, dst_v, acc_v):
    c = lax.axis_index("c")
    s = lax.axis_index("s")

    @pl.when(jnp.logical_and(c == 0, s == 0))
    def _():
        pltpu.sync_copy(edges_hbm.at[0], src_v)
        pltpu.sync_copy(edges_hbm.at[1], dst_v)

        def zero_body(i, carry):
            acc_v[pl.ds(i * _LANES, _LANES)] = jnp.zeros((_LANES,),
                                                         jnp.float32)
            return carry

        lax.fori_loop(0, (_N * _N) // _LANES, zero_body, 0)

        ones = jnp.full((_LANES,), 1.0, jnp.float32)
        lane = lax.iota(jnp.int32, _LANES)

        def edge_body(i, carry):
            sv = src_v[pl.ds(i * _LANES, _LANES)]
            dv = dst_v[pl.ds(i * _LANES, _LANES)]
            valid = (i * _LANES + lane) < _NEDGE
            plsc.addupdate_scatter(acc_v, [dv * _N + sv], ones, mask=valid)
            return carry

        nvec = (_NEDGE + _LANES - 1) // _LANES
        lax.fori_loop(0, nvec, edge_body, 0)

        def loop_body(i, carry):
            node = i * _LANES + lane
            valid = node < _N
            plsc.addupdate_scatter(acc_v, [node * (_N + 1)], ones, mask=valid)
            return carry

        lax.fori_loop(0, (_N + _LANES - 1) // _LANES, loop_body, 0)
        pltpu.sync_copy(acc_v, out_hbm)


_build_counts = functools.partial(
    pl.kernel,
    mesh=plsc.VectorSubcoreMesh(core_axis_name="c", subcore_axis_name="s"),
    out_type=jax.ShapeDtypeStruct((_N * _N,), jnp.float32),
    compiler_params=pltpu.CompilerParams(needs_layout_passes=False),
    scratch_types=[
        pltpu.VMEM((_NEDGE + 4,), jnp.int32),
        pltpu.VMEM((_NEDGE + 4,), jnp.int32),
        pltpu.VMEM((_N * _N,), jnp.float32),
    ],
)(_counts_body)


def _mm(a, b, ca, cb):
    return lax.dot_general(a, b, (((ca,), (cb,)), ((), ())),
                           precision=lax.Precision.HIGHEST,
                           preferred_element_type=jnp.float32)


def _node_mask(rows):
    r_id = lax.broadcasted_iota(jnp.int32, (rows, _N), 0)
    c_id = lax.broadcasted_iota(jnp.int32, (rows, _N), 1)
    return (lax.rem(r_id, _N) == c_id).astype(jnp.float32)


def _stripe_mask(nb, rows):
    q_id = lax.broadcasted_iota(jnp.int32, (nb, rows), 0)
    r_id = lax.broadcasted_iota(jnp.int32, (nb, rows), 1)
    return ((r_id // _N) == q_id).astype(jnp.float32)


def _fused_body(x_ref, emb_ref, c_ref, wlin_ref, ai_ref, aj_ref, aei_ref,
                aej_ref, bias_ref, g1_ref, b1_ref, g2_ref, b2_ref, ow_ref,
                obn_ref, o_ref, out_s):
    wlin = wlin_ref[:]
    att_i = ai_ref[:]
    att_j = aj_ref[:]
    bias = bias_ref[:]

    nm16 = _node_mask(_CHN)
    nm4 = nm16[:_GN, :]
    sel4 = _stripe_mask(_G, _GN)
    sel16 = _stripe_mask(_CH, _CHN)

    emb16 = _mm(nm16, emb_ref[:], 1, 0)
    emb4 = emb16[:_GN, :]
    c4 = _mm(nm4, c_ref[:], 1, 0)
    ei = _mm(emb4, aei_ref[:], 1, 1)
    ej = _mm(emb4, aej_ref[:], 1, 1)

    zstat = jnp.zeros((1, _D), jnp.float32)

    def _one_group(base):
        wx_g = _mm(x_ref[pl.ds(base, _GN), :], wlin, 1, 1)
        ai = _mm(wx_g, att_i, 1, 1) + ei
        aj_c = _mm(wx_g, att_j, 1, 1) + ej
        ajg = _mm(sel4, aj_c * nm4, 1, 0)
        aj_rep = _mm(sel4, ajg, 0, 0)
        alpha = ai + aj_rep
        alpha = jnp.where(alpha >= 0, alpha, 0.2 * alpha)
        am = jnp.max(jnp.where(c4 > 0, alpha, -1e30), axis=1, keepdims=True)
        ex = c4 * jnp.exp(jnp.minimum(alpha - am, 0.0))
        den = jnp.sum(ex, axis=1, keepdims=True)
        w = ex / (den + 1e-16)
        return wx_g, w

    _UNROLL = 4

    def group_body(g, carry):
        s1, s2 = carry
        parts = [_one_group((_UNROLL * g + u) * _GN) for u in range(_UNROLL)]
        for u, (wx_g, w) in enumerate(parts):
            base = (_UNROLL * g + u) * _GN
            out_g = jnp.concatenate(
                [_mm(w[q * _N:(q + 1) * _N, :],
                     wx_g[q * _N:(q + 1) * _N, :], 1, 0) for q in range(_G)],
                axis=0) + bias
            out_s[pl.ds(base, _GN), :] = out_g
            s1 = s1 + jnp.sum(out_g, axis=0, keepdims=True)
            s2 = s2 + jnp.sum(out_g * out_g, axis=0, keepdims=True)
        return s1, s2

    s1, s2 = lax.fori_loop(0, _NGROUPS // _UNROLL, group_body, (zstat, zstat))

    n_rows = float(_B * _N)
    mu1 = s1 / n_rows
    var1 = s2 / n_rows - mu1 * mu1
    scale1 = lax.rsqrt(var1 + 1e-5) * g1_ref[:]
    b1 = b1_ref[:]

    def bn1_body(i, carry):
        t1, t2 = carry
        base = i * _CHN
        o1 = (out_s[pl.ds(base, _CHN), :] - mu1) * scale1 + b1
        h = jnp.maximum(o1, 0.0) * emb16
        out_s[pl.ds(base, _CHN), :] = h
        t1 = t1 + jnp.sum(h, axis=0, keepdims=True)
        t2 = t2 + jnp.sum(h * h, axis=0, keepdims=True)
        return t1, t2

    t1, t2 = lax.fori_loop(0, _NCHUNK, bn1_body, (zstat, zstat))

    mu2 = t1 / n_rows
    var2 = t2 / n_rows - mu2 * mu2
    scale2 = lax.rsqrt(var2 + 1e-5) * g2_ref[:]
    b2 = b2_ref[:]
    ow = ow_ref[:]
    obn = obn_ref[:]

    def bn2_body(i, carry):
        base = i * _CHN
        h2 = jnp.maximum((out_s[pl.ds(base, _CHN), :] - mu2) * scale2 + b2,
                         0.0)
        oc = _mm(h2, ow, 1, 1)
        orows = _mm(sel16, oc * nm16, 1, 0)
        o_ref[pl.ds(i * _CH, _CH), :] = orows + obn
        return carry

    lax.fori_loop(0, _NCHUNK, bn2_body, 0)


_fused = pl.pallas_call(
    _fused_body,
    out_shape=jax.ShapeDtypeStruct((_B, _N), jnp.float32),
    scratch_shapes=[
        pltpu.VMEM((_B * _N, _D), jnp.float32),
    ],
)


def kernel(data, labels, org_edge_index, emb_table, W_lin, att_i, att_j,
           att_em_i, att_em_j, gnn_bias, bn1_gamma, bn1_beta, bn2_gamma,
           bn2_beta, out_W, out_b):
    B, N, F = data.shape

    edges = jnp.pad(org_edge_index, ((0, 0), (0, 4)))
    counts = _build_counts(edges)
    c_mat = counts.reshape(N, N)

    x2 = data.reshape(B * N, F)
    obn = jnp.broadcast_to(out_b.reshape(1, 1), (1, N))

    return _fused(x2, emb_table, c_mat, W_lin,
                  att_i.reshape(1, -1), att_j.reshape(1, -1),
                  att_em_i.reshape(1, -1), att_em_j.reshape(1, -1),
                  gnn_bias.reshape(1, -1), bn1_gamma.reshape(1, -1),
                  bn1_beta.reshape(1, -1), bn2_gamma.reshape(1, -1),
                  bn2_beta.reshape(1, -1), out_W.reshape(1, -1), obn)


# speedup vs baseline: 76.6180x; 1.0265x over previous
import functools

import jax
import jax.numpy as jnp
from jax import lax
from jax.experimental import pallas as pl
from jax.experimental.pallas import tpu as pltpu
from jax.experimental.pallas import tpu_sc as plsc

_B, _N, _F, _D = 128, 100, 64, 64
_G = 4
_GN = _G * _N
_NGROUPS = _B // _G
_CH = 16
_CHN = _CH * _N
_NCHUNK = _B // _CH
_NEDGE = 1500
_LANES = 16


def _counts_body(edges_hbm, out_hbm, src_v, dst_v, acc_v):
    c = lax.axis_index("c")
    s = lax.axis_index("s")

    @pl.when(jnp.logical_and(c == 0, s == 0))
    def _():
        pltpu.sync_copy(edges_hbm.at[0], src_v)
        pltpu.sync_copy(edges_hbm.at[1], dst_v)

        def zero_body(i, carry):
            acc_v[pl.ds(i * _LANES, _LANES)] = jnp.zeros((_LANES,),
                                                         jnp.float32)
            return carry

        lax.fori_loop(0, (_N * _N) // _LANES, zero_body, 0)

        ones = jnp.full((_LANES,), 1.0, jnp.float32)
        lane = lax.iota(jnp.int32, _LANES)

        def edge_body(i, carry):
            sv = src_v[pl.ds(i * _LANES, _LANES)]
            dv = dst_v[pl.ds(i * _LANES, _LANES)]
            valid = (i * _LANES + lane) < _NEDGE
            plsc.addupdate_scatter(acc_v, [dv * _N + sv], ones, mask=valid)
            return carry

        nvec = (_NEDGE + _LANES - 1) // _LANES
        lax.fori_loop(0, nvec, edge_body, 0)

        def loop_body(i, carry):
            node = i * _LANES + lane
            valid = node < _N
            plsc.addupdate_scatter(acc_v, [node * (_N + 1)], ones, mask=valid)
            return carry

        lax.fori_loop(0, (_N + _LANES - 1) // _LANES, loop_body, 0)
        pltpu.sync_copy(acc_v, out_hbm)


_build_counts = functools.partial(
    pl.kernel,
    mesh=plsc.VectorSubcoreMesh(core_axis_name="c", subcore_axis_name="s"),
    out_type=jax.ShapeDtypeStruct((_N * _N,), jnp.float32),
    compiler_params=pltpu.CompilerParams(needs_layout_passes=False),
    scratch_types=[
        pltpu.VMEM((_NEDGE + 4,), jnp.int32),
        pltpu.VMEM((_NEDGE + 4,), jnp.int32),
        pltpu.VMEM((_N * _N,), jnp.float32),
    ],
)(_counts_body)


def _mm(a, b, ca, cb):
    return lax.dot_general(a, b, (((ca,), (cb,)), ((), ())),
                           precision=lax.Precision.HIGHEST,
                           preferred_element_type=jnp.float32)


def _node_mask(rows):
    r_id = lax.broadcasted_iota(jnp.int32, (rows, _N), 0)
    c_id = lax.broadcasted_iota(jnp.int32, (rows, _N), 1)
    return (lax.rem(r_id, _N) == c_id).astype(jnp.float32)


def _stripe_mask(nb, rows):
    q_id = lax.broadcasted_iota(jnp.int32, (nb, rows), 0)
    r_id = lax.broadcasted_iota(jnp.int32, (nb, rows), 1)
    return ((r_id // _N) == q_id).astype(jnp.float32)


def _fused_body(x_ref, emb_ref, c_ref, wlin_ref, ai_ref, aj_ref, aei_ref,
                aej_ref, bias_ref, g1_ref, b1_ref, g2_ref, b2_ref, ow_ref,
                obn_ref, o_ref, out_s):
    wlin = wlin_ref[:]
    att_i = ai_ref[:]
    att_j = aj_ref[:]
    bias = bias_ref[:]

    nm16 = _node_mask(_CHN)
    nm4 = nm16[:_GN, :]
    sel4 = _stripe_mask(_G, _GN)
    sel16 = _stripe_mask(_CH, _CHN)

    emb16 = _mm(nm16, emb_ref[:], 1, 0)
    emb4 = emb16[:_GN, :]
    c4 = _mm(nm4, c_ref[:], 1, 0)
    ei = _mm(emb4, aei_ref[:], 1, 1)
    ej = _mm(emb4, aej_ref[:], 1, 1)

    zstat = jnp.zeros((1, _D), jnp.float32)

    def _one_group(base):
        wx_g = _mm(x_ref[pl.ds(base, _GN), :], wlin, 1, 1)
        ai = _mm(wx_g, att_i, 1, 1) + ei
        aj_c = _mm(wx_g, att_j, 1, 1) + ej
        ajg = _mm(sel4, aj_c * nm4, 1, 0)
        aj_rep = _mm(sel4, ajg, 0, 0)
        alpha = ai + aj_rep
        alpha = jnp.where(alpha >= 0, alpha, 0.2 * alpha)
        am = jnp.max(jnp.where(c4 > 0, alpha, -1e30), axis=1, keepdims=True)
        ex = c4 * jnp.exp(jnp.minimum(alpha - am, 0.0))
        den = jnp.sum(ex, axis=1, keepdims=True)
        w = ex / (den + 1e-16)
        return wx_g, w

    _UNROLL = 8

    def group_body(g, carry):
        s1, s2 = carry
        parts = [_one_group((_UNROLL * g + u) * _GN) for u in range(_UNROLL)]
        for u, (wx_g, w) in enumerate(parts):
            base = (_UNROLL * g + u) * _GN
            out_g = jnp.concatenate(
                [_mm(w[q * _N:(q + 1) * _N, :],
                     wx_g[q * _N:(q + 1) * _N, :], 1, 0) for q in range(_G)],
                axis=0) + bias
            out_s[pl.ds(base, _GN), :] = out_g
            s1 = s1 + jnp.sum(out_g, axis=0, keepdims=True)
            s2 = s2 + jnp.sum(out_g * out_g, axis=0, keepdims=True)
        return s1, s2

    s1, s2 = lax.fori_loop(0, _NGROUPS // _UNROLL, group_body, (zstat, zstat))

    n_rows = float(_B * _N)
    mu1 = s1 / n_rows
    var1 = s2 / n_rows - mu1 * mu1
    scale1 = lax.rsqrt(var1 + 1e-5) * g1_ref[:]
    b1 = b1_ref[:]

    def bn1_body(i, carry):
        t1, t2 = carry
        base = i * _CHN
        o1 = (out_s[pl.ds(base, _CHN), :] - mu1) * scale1 + b1
        h = jnp.maximum(o1, 0.0) * emb16
        out_s[pl.ds(base, _CHN), :] = h
        t1 = t1 + jnp.sum(h, axis=0, keepdims=True)
        t2 = t2 + jnp.sum(h * h, axis=0, keepdims=True)
        return t1, t2

    t1, t2 = lax.fori_loop(0, _NCHUNK, bn1_body, (zstat, zstat))

    mu2 = t1 / n_rows
    var2 = t2 / n_rows - mu2 * mu2
    scale2 = lax.rsqrt(var2 + 1e-5) * g2_ref[:]
    b2 = b2_ref[:]
    ow = ow_ref[:]
    obn = obn_ref[:]

    def bn2_body(i, carry):
        base = i * _CHN
        h2 = jnp.maximum((out_s[pl.ds(base, _CHN), :] - mu2) * scale2 + b2,
                         0.0)
        oc = _mm(h2, ow, 1, 1)
        orows = _mm(sel16, oc * nm16, 1, 0)
        o_ref[pl.ds(i * _CH, _CH), :] = orows + obn
        return carry

    lax.fori_loop(0, _NCHUNK, bn2_body, 0)


_fused = pl.pallas_call(
    _fused_body,
    out_shape=jax.ShapeDtypeStruct((_B, _N), jnp.float32),
    scratch_shapes=[
        pltpu.VMEM((_B * _N, _D), jnp.float32),
    ],
)


def kernel(data, labels, org_edge_index, emb_table, W_lin, att_i, att_j,
           att_em_i, att_em_j, gnn_bias, bn1_gamma, bn1_beta, bn2_gamma,
           bn2_beta, out_W, out_b):
    B, N, F = data.shape

    edges = jnp.pad(org_edge_index, ((0, 0), (0, 4)))
    counts = _build_counts(edges)
    c_mat = counts.reshape(N, N)

    x2 = data.reshape(B * N, F)
    obn = jnp.broadcast_to(out_b.reshape(1, 1), (1, N))

    return _fused(x2, emb_table, c_mat, W_lin,
                  att_i.reshape(1, -1), att_j.reshape(1, -1),
                  att_em_i.reshape(1, -1), att_em_j.reshape(1, -1),
                  gnn_bias.reshape(1, -1), bn1_gamma.reshape(1, -1),
                  bn1_beta.reshape(1, -1), bn2_gamma.reshape(1, -1),
                  bn2_beta.reshape(1, -1), out_W.reshape(1, -1), obn)
